# baseline (speedup 1.0000x reference)
import functools

import jax
import jax.numpy as jnp
from jax import lax
from jax.experimental import pallas as pl
from jax.experimental.pallas import tpu as pltpu
from jax.experimental.pallas import tpu_sc as plsc

_SCALE = 10.0


@functools.cache
def _make_sc_lookup(B, V2, D2):
    D = D2 // 2
    info = plsc.get_sparse_core_info()
    NC, NS, L = info.num_cores, info.num_subcores, info.num_lanes
    NW = NC * NS
    assert B % (8 * NW) == 0 and D % L == 0
    b_per_w = B // NW
    mesh = plsc.VectorSubcoreMesh(core_axis_name="c", subcore_axis_name="s")

    @functools.partial(
        pl.kernel,
        mesh=mesh,
        out_type=jax.ShapeDtypeStruct((B // 2, D2), jnp.float32),
        scratch_types=[
            pltpu.VMEM((b_per_w,), jnp.int32),
            pltpu.VMEM((b_per_w,), jnp.int32),
            pltpu.VMEM((b_per_w, D2), jnp.float32),
            pltpu.VMEM((b_per_w // 2, D2), jnp.float32),
            pltpu.SemaphoreType.DMA,
        ],
    )
    def lookup(idx_hbm, table_hbm, out_hbm, idx_v, pair_v, rows_v, out_v, sem):
        wid = lax.axis_index("s") * NC + lax.axis_index("c")
        base = wid * b_per_w
        pltpu.sync_copy(idx_hbm.at[pl.ds(base, b_per_w)], idx_v)

        def to_pairs(k, carry):
            vec = idx_v[pl.ds(k * L, L)]
            pair_v[pl.ds(k * L, L)] = lax.shift_right_logical(vec, 1)
            return carry

        lax.fori_loop(0, b_per_w // L, to_pairs, None)
        pltpu.async_copy(table_hbm.at[pair_v], rows_v, sem).wait()

        nj = D // L

        def select_scale(k, carry):
            vec = idx_v[pl.ds(k * L, L)]
            for u in range(L):
                half = lax.bitwise_and(vec[u], 1)
                off = half * D
                src_row = k * L + u
                dst_row = k * (L // 2) + u // 2
                dst_off = (u % 2) * D
                for j in range(nj):
                    out_v[dst_row, pl.ds(dst_off + j * L, L)] = (
                        rows_v[src_row, pl.ds(off + j * L, L)] * _SCALE
                    )
            return carry

        lax.fori_loop(0, b_per_w // L, select_scale, None)
        pltpu.sync_copy(out_v, out_hbm.at[pl.ds(wid * (b_per_w // 2), b_per_w // 2)])

    return lookup


def kernel(x, weight):
    (B,) = x.shape
    V, D = weight.shape
    table2 = weight.reshape(V // 2, 2 * D)
    fn = _make_sc_lookup(B, V // 2, 2 * D)
    out2 = fn(x.astype(jnp.int32), table2)
    return out2.reshape(B, D)


# speedup vs baseline: 2.2354x; 2.2354x over previous
import functools

import jax
import jax.numpy as jnp
from jax import lax
from jax.experimental import pallas as pl
from jax.experimental.pallas import tpu as pltpu
from jax.experimental.pallas import tpu_sc as plsc

_SCALE = 10.0


@functools.cache
def _make_sc_lookup(B, V, D):
    info = plsc.get_sparse_core_info()
    NC, NS, L = info.num_cores, info.num_subcores, info.num_lanes
    NW = NC * NS
    assert D % NW == 0 and B % L == 0
    rows_per_w = D // NW
    C = 8192
    n_chunks = B // C
    assert B % C == 0
    mesh = plsc.VectorSubcoreMesh(core_axis_name="c", subcore_axis_name="s")

    @functools.partial(
        pl.kernel,
        mesh=mesh,
        out_type=jax.ShapeDtypeStruct((D, B), jnp.float32),
        compiler_params=pltpu.CompilerParams(needs_layout_passes=False),
        scratch_types=[
            pltpu.VMEM((V,), jnp.float32),
            pltpu.VMEM((C,), jnp.int32),
            pltpu.VMEM((C,), jnp.float32),
            pltpu.SemaphoreType.DMA,
        ],
    )
    def lookup(idx_hbm, table_hbm, out_hbm, row_v, idx_v, obuf_v, sem):
        wid = lax.axis_index("s") * NC + lax.axis_index("c")

        def do_row(r, carry):
            d = wid * rows_per_w + r
            pltpu.sync_copy(table_hbm.at[d], row_v)

            def do_chunk(c, carry2):
                pltpu.sync_copy(idx_hbm.at[pl.ds(c * C, C)], idx_v)

                def gather16(k, carry3):
                    for u in range(4):
                        sl = pl.ds((k * 4 + u) * L, L)
                        iv = idx_v[sl]
                        obuf_v[sl] = plsc.load_gather(row_v, [iv]) * _SCALE
                    return carry3

                lax.fori_loop(0, C // (4 * L), gather16, None)
                pltpu.sync_copy(obuf_v, out_hbm.at[d, pl.ds(c * C, C)])
                return carry2

            lax.fori_loop(0, n_chunks, do_chunk, None)
            return carry

        lax.fori_loop(0, rows_per_w, do_row, None)

    return lookup


def kernel(x, weight):
    (B,) = x.shape
    V, D = weight.shape
    fn = _make_sc_lookup(B, V, D)
    outT = fn(x.astype(jnp.int32), weight.T)
    return outT.T


# speedup vs baseline: 3.0562x; 1.3672x over previous
import functools

import jax
import jax.numpy as jnp
from jax import lax
from jax.experimental import pallas as pl
from jax.experimental.pallas import tpu as pltpu
from jax.experimental.pallas import tpu_sc as plsc

_SCALE = 10.0


@functools.cache
def _make_sc_lookup(B, V, D):
    info = plsc.get_sparse_core_info()
    NC, NS, L = info.num_cores, info.num_subcores, info.num_lanes
    NW = NC * NS
    assert D % NW == 0 and B % L == 0
    rows_per_w = D // NW
    C = 4096
    n_chunks = B // C
    assert B % C == 0
    mesh = plsc.VectorSubcoreMesh(core_axis_name="c", subcore_axis_name="s")

    @functools.partial(
        pl.kernel,
        mesh=mesh,
        out_type=jax.ShapeDtypeStruct((D, B), jnp.float32),
        compiler_params=pltpu.CompilerParams(needs_layout_passes=False),
        scratch_types=[
            pltpu.VMEM((V,), jnp.float32),
            pltpu.VMEM((B,), jnp.int32),
            pltpu.VMEM((C,), jnp.float32),
            pltpu.VMEM((C,), jnp.float32),
            pltpu.SemaphoreType.DMA,
            pltpu.SemaphoreType.DMA,
            pltpu.SemaphoreType.DMA,
        ],
    )
    def lookup(idx_hbm, table_hbm, out_hbm, row_v, idx_v, ob0, ob1, rsem,
               ws0, ws1):
        wid = lax.axis_index("s") * NC + lax.axis_index("c")
        obufs = (ob0, ob1)
        wsems = (ws0, ws1)

        pltpu.async_copy(idx_hbm, idx_v, rsem)
        rwait = pltpu.make_async_copy(idx_hbm, idx_v, rsem)
        rwait.wait()

        for r in range(rows_per_w):
            d = wid * rows_per_w + r
            pltpu.sync_copy(table_hbm.at[d], row_v)
            for c in range(n_chunks):
                buf = obufs[c % 2]
                sem = wsems[c % 2]
                if r > 0 or c >= 2:
                    pltpu.make_async_copy(
                        out_hbm.at[d, pl.ds(c * C, C)], buf, sem
                    ).wait()

                @plsc.parallel_loop(0, C // L, unroll=8)
                def gather16(k):
                    sl = pl.ds(k * L, L)
                    iv = idx_v[pl.ds(c * C + k * L, L)]
                    buf[sl] = plsc.load_gather(row_v, [iv]) * _SCALE

                pltpu.async_copy(
                    buf, out_hbm.at[d, pl.ds(c * C, C)], sem
                )
        pltpu.make_async_copy(out_hbm.at[0, pl.ds(0, C)], ob0, ws0).wait()
        pltpu.make_async_copy(out_hbm.at[0, pl.ds(0, C)], ob1, ws1).wait()

    return lookup


def kernel(x, weight):
    (B,) = x.shape
    V, D = weight.shape
    fn = _make_sc_lookup(B, V, D)
    outT = fn(x.astype(jnp.int32), weight.T)
    return outT.T


# speedup vs baseline: 3.0933x; 1.0121x over previous
import functools

import jax
import jax.numpy as jnp
from jax import lax
from jax.experimental import pallas as pl
from jax.experimental.pallas import tpu as pltpu
from jax.experimental.pallas import tpu_sc as plsc

_SCALE = 10.0


@functools.cache
def _make_sc_lookup(B, V, D):
    info = plsc.get_sparse_core_info()
    NC, NS, L = info.num_cores, info.num_subcores, info.num_lanes
    NW = NC * NS
    assert D % NW == 0 and B % L == 0
    rows_per_w = D // NW
    C = 4096
    n_chunks = B // C
    assert B % C == 0
    mesh = plsc.VectorSubcoreMesh(core_axis_name="c", subcore_axis_name="s")

    @functools.partial(
        pl.kernel,
        mesh=mesh,
        out_type=jax.ShapeDtypeStruct((D, B), jnp.float32),
        compiler_params=pltpu.CompilerParams(
            needs_layout_passes=False,
            disable_bounds_checks=True,
            disable_semaphore_checks=True,
        ),
        scratch_types=[
            pltpu.VMEM((V,), jnp.float32),
            pltpu.VMEM((B,), jnp.int32),
            pltpu.VMEM((C,), jnp.float32),
            pltpu.VMEM((C,), jnp.float32),
            pltpu.SemaphoreType.DMA,
            pltpu.SemaphoreType.DMA,
            pltpu.SemaphoreType.DMA,
        ],
    )
    def lookup(idx_hbm, table_hbm, out_hbm, row_v, idx_v, ob0, ob1, rsem,
               ws0, ws1):
        wid = lax.axis_index("s") * NC + lax.axis_index("c")
        obufs = (ob0, ob1)
        wsems = (ws0, ws1)

        pltpu.async_copy(idx_hbm, idx_v, rsem)
        idx_waited = False

        for r in range(rows_per_w):
            d = wid * rows_per_w + r
            pltpu.sync_copy(table_hbm.at[d], row_v)
            if not idx_waited:
                pltpu.make_async_copy(idx_hbm, idx_v, rsem).wait()
                idx_waited = True
            for c in range(n_chunks):
                buf = obufs[c % 2]
                sem = wsems[c % 2]
                if r > 0 or c >= 2:
                    pltpu.make_async_copy(
                        out_hbm.at[d, pl.ds(c * C, C)], buf, sem
                    ).wait()

                @plsc.parallel_loop(0, C // L, unroll=8)
                def gather16(k):
                    sl = pl.ds(k * L, L)
                    iv = idx_v[pl.ds(c * C + k * L, L)]
                    buf[sl] = plsc.load_gather(row_v, [iv]) * _SCALE

                pltpu.async_copy(
                    buf, out_hbm.at[d, pl.ds(c * C, C)], sem
                )
        pltpu.make_async_copy(out_hbm.at[0, pl.ds(0, C)], ob0, ws0).wait()
        pltpu.make_async_copy(out_hbm.at[0, pl.ds(0, C)], ob1, ws1).wait()

    return lookup


def kernel(x, weight):
    (B,) = x.shape
    V, D = weight.shape
    fn = _make_sc_lookup(B, V, D)
    outT = fn(x.astype(jnp.int32), weight.T)
    return outT.T


# speedup vs baseline: 3.1116x; 1.0059x over previous
import functools

import jax
import jax.numpy as jnp
from jax import lax
from jax.experimental import pallas as pl
from jax.experimental.pallas import tpu as pltpu
from jax.experimental.pallas import tpu_sc as plsc

_SCALE = 10.0


@functools.cache
def _make_sc_lookup(B, V, D):
    info = plsc.get_sparse_core_info()
    NC, NS, L = info.num_cores, info.num_subcores, info.num_lanes
    NW = NC * NS
    assert D % NW == 0 and B % L == 0
    rows_per_w = D // NW
    C = 4096
    n_chunks = B // C
    assert B % C == 0
    mesh = plsc.VectorSubcoreMesh(core_axis_name="c", subcore_axis_name="s")

    @functools.partial(
        pl.kernel,
        mesh=mesh,
        out_type=jax.ShapeDtypeStruct((D, B), jnp.float32),
        compiler_params=pltpu.CompilerParams(
            needs_layout_passes=False,
            disable_bounds_checks=True,
            disable_semaphore_checks=True,
            skip_device_barrier=True,
        ),
        scratch_types=[
            pltpu.VMEM((V,), jnp.float32),
            pltpu.VMEM((B,), jnp.int32),
            pltpu.VMEM((C,), jnp.float32),
            pltpu.VMEM((C,), jnp.float32),
            pltpu.SemaphoreType.DMA,
            pltpu.SemaphoreType.DMA,
            pltpu.SemaphoreType.DMA,
        ],
    )
    def lookup(idx_hbm, table_hbm, out_hbm, row_v, idx_v, ob0, ob1, rsem,
               ws0, ws1):
        wid = lax.axis_index("s") * NC + lax.axis_index("c")
        obufs = (ob0, ob1)
        wsems = (ws0, ws1)

        pltpu.async_copy(idx_hbm, idx_v, rsem)
        idx_waited = False

        for r in range(rows_per_w):
            d = wid * rows_per_w + r
            pltpu.sync_copy(table_hbm.at[d], row_v)
            if not idx_waited:
                pltpu.make_async_copy(idx_hbm, idx_v, rsem).wait()
                idx_waited = True
            for c in range(n_chunks):
                buf = obufs[c % 2]
                sem = wsems[c % 2]
                if r > 0 or c >= 2:
                    pltpu.make_async_copy(
                        out_hbm.at[d, pl.ds(c * C, C)], buf, sem
                    ).wait()

                @plsc.parallel_loop(0, C // L, unroll=8)
                def gather16(k):
                    sl = pl.ds(k * L, L)
                    iv = idx_v[pl.ds(c * C + k * L, L)]
                    buf[sl] = plsc.load_gather(row_v, [iv]) * _SCALE

                pltpu.async_copy(
                    buf, out_hbm.at[d, pl.ds(c * C, C)], sem
                )
        pltpu.make_async_copy(out_hbm.at[0, pl.ds(0, C)], ob0, ws0).wait()
        pltpu.make_async_copy(out_hbm.at[0, pl.ds(0, C)], ob1, ws1).wait()

    return lookup


def kernel(x, weight):
    (B,) = x.shape
    V, D = weight.shape
    fn = _make_sc_lookup(B, V, D)
    outT = fn(x.astype(jnp.int32), weight.T)
    return outT.T


# speedup vs baseline: 3.1131x; 1.0005x over previous
import functools

import jax
import jax.numpy as jnp
from jax import lax
from jax.experimental import pallas as pl
from jax.experimental.pallas import tpu as pltpu
from jax.experimental.pallas import tpu_sc as plsc

_SCALE = 10.0


@functools.cache
def _make_sc_lookup(B, V, D):
    info = plsc.get_sparse_core_info()
    NC, NS, L = info.num_cores, info.num_subcores, info.num_lanes
    NW = NC * NS
    assert D % NW == 0 and B % L == 0
    rows_per_w = D // NW
    C = 4096
    n_chunks = B // C
    assert B % C == 0
    mesh = plsc.VectorSubcoreMesh(core_axis_name="c", subcore_axis_name="s")

    @functools.partial(
        pl.kernel,
        mesh=mesh,
        out_type=jax.ShapeDtypeStruct((D, B), jnp.float32),
        compiler_params=pltpu.CompilerParams(
            needs_layout_passes=False,
            disable_bounds_checks=True,
            disable_semaphore_checks=True,
        ),
        scratch_types=[
            pltpu.VMEM((V,), jnp.float32),
            pltpu.VMEM((B,), jnp.int32),
            pltpu.VMEM((C,), jnp.float32),
            pltpu.VMEM((C,), jnp.float32),
            pltpu.SemaphoreType.DMA,
            pltpu.SemaphoreType.DMA,
            pltpu.SemaphoreType.DMA,
            pltpu.SemaphoreType.DMA,
        ],
    )
    def lookup(idx_hbm, table_hbm, out_hbm, row_v, idx_v, ob0, ob1, rsem,
               isem, ws0, ws1):
        wid = lax.axis_index("s") * NC + lax.axis_index("c")
        obufs = (ob0, ob1)
        wsems = (ws0, ws1)

        pltpu.async_copy(idx_hbm, idx_v, isem)
        idx_waited = False

        for r in range(rows_per_w):
            d = wid * rows_per_w + r
            pltpu.async_copy(table_hbm.at[d], row_v, rsem)
            pltpu.make_async_copy(table_hbm.at[d], row_v, rsem).wait()
            if not idx_waited:
                pltpu.make_async_copy(idx_hbm, idx_v, isem).wait()
                idx_waited = True
            for c in range(n_chunks):
                buf = obufs[c % 2]
                sem = wsems[c % 2]
                if r > 0 or c >= 2:
                    pltpu.make_async_copy(
                        out_hbm.at[d, pl.ds(c * C, C)], buf, sem
                    ).wait()

                @plsc.parallel_loop(0, C // L, unroll=8)
                def gather16(k):
                    sl = pl.ds(k * L, L)
                    iv = idx_v[pl.ds(c * C + k * L, L)]
                    buf[sl] = plsc.load_gather(row_v, [iv]) * _SCALE

                pltpu.async_copy(
                    buf, out_hbm.at[d, pl.ds(c * C, C)], sem
                )
        pltpu.make_async_copy(out_hbm.at[0, pl.ds(0, C)], ob0, ws0).wait()
        pltpu.make_async_copy(out_hbm.at[0, pl.ds(0, C)], ob1, ws1).wait()

    return lookup


def kernel(x, weight):
    (B,) = x.shape
    V, D = weight.shape
    fn = _make_sc_lookup(B, V, D)
    outT = fn(x.astype(jnp.int32), weight.T)
    return outT.T
